# Initial kernel scaffold; baseline (speedup 1.0000x reference)
import jax
import jax.numpy as jnp
from jax.experimental import pallas as pl


def kernel(x, edge_index, edge_type, W1_rel, W1_root, b1, W2_rel, W2_root, b2):
    raise NotImplementedError("write your pallas kernel here")


# speedup vs baseline: 5.6582x; 5.6582x over previous
import functools

import jax
import jax.numpy as jnp
from jax import lax
from jax.experimental import pallas as pl
from jax.experimental.pallas import tpu as pltpu
from jax.experimental.pallas import tpu_sc as plsc

N = 10000
E = 320000
IN = 128
H = 128
OUT = 2

NC = 2
NS = 16
NW = NC * NS
N_PAD = 10240
RPT = N_PAD // NS
EPW = E // NW
K = 80
NCHUNK = EPW // K
NB = 1024
G = N_PAD // NB


def _make_sc_agg(with_deg):
  mesh = plsc.VectorSubcoreMesh(core_axis_name="c", subcore_axis_name="s")
  out_type = [jax.ShapeDtypeStruct((NC * N_PAD, H), jnp.float32)]
  if with_deg:
    out_type.append(jax.ShapeDtypeStruct((NW, N_PAD), jnp.float32))
  scratch = [
      pltpu.VMEM((K,), jnp.int32),
      pltpu.VMEM((K,), jnp.int32),
      pltpu.VMEM((K, H), jnp.float32),
      pltpu.VMEM_SHARED((N_PAD, H), jnp.float32),
  ]
  if with_deg:
    scratch.append(pltpu.VMEM((N_PAD,), jnp.float32))
  NSEG = RPT // K

  def body(feat, src, dst, zeros_f, zeros_deg, *refs):
    if with_deg:
      (agg_out, deg_out, src_v, dst_v, rows_v, acc_s, deg_v) = refs
    else:
      (agg_out, src_v, dst_v, rows_v, acc_s) = refs
    c = lax.axis_index("c")
    s = lax.axis_index("s")
    w = c * NS + s
    row0 = s * RPT

    pltpu.sync_copy(zeros_f, rows_v)
    for i in range(NSEG):
      pltpu.sync_copy(rows_v, acc_s.at[pl.ds(row0 + i * K, K)])
    if with_deg:
      pltpu.sync_copy(zeros_deg, deg_v)
    plsc.subcore_barrier()

    ones16 = jnp.full((16,), 1.0, jnp.float32)

    def chunk(j, carry):
      base = w * EPW + j * K
      pltpu.sync_copy(src.at[pl.ds(base, K)], src_v)
      pltpu.sync_copy(dst.at[pl.ds(base, K)], dst_v)
      pltpu.sync_copy(feat.at[src_v], rows_v)
      pltpu.sync_copy(rows_v, acc_s.at[dst_v], add=True)
      if with_deg:
        for i in range(K // 16):
          dst16 = dst_v[pl.ds(i * 16, 16)]
          plsc.addupdate_scatter(deg_v, [dst16], ones16)
      return carry

    lax.fori_loop(0, NCHUNK, chunk, 0)
    plsc.subcore_barrier()

    out0 = c * N_PAD + row0
    for i in range(NSEG):
      pltpu.sync_copy(acc_s.at[pl.ds(row0 + i * K, K)], rows_v)
      pltpu.sync_copy(rows_v, agg_out.at[pl.ds(out0 + i * K, K)])
    if with_deg:
      pltpu.sync_copy(deg_v, deg_out.at[w])

  return pl.kernel(
      body, out_type=out_type, mesh=mesh, scratch_types=scratch,
      compiler_params=pltpu.CompilerParams(needs_layout_passes=False))


@functools.lru_cache(maxsize=None)
def _sc_agg_fn(with_deg):
  return _make_sc_agg(with_deg)


def _sc_agg_deg(*args):
  return _sc_agg_fn(True)(*args)


def _sc_agg(*args):
  return _sc_agg_fn(False)(*args)[0]


def _tc_layer1_body(x_ref, aggp_ref, degp_ref, wr_ref, wl_ref, b_ref, o_ref):
  agg = aggp_ref[0] + aggp_ref[1]
  deg = jnp.sum(degp_ref[...], axis=1, keepdims=True)
  inv = 1.0 / jnp.maximum(deg, 1.0)
  z = jnp.dot(x_ref[...], wr_ref[...], preferred_element_type=jnp.float32)
  z = z + jnp.dot(agg, wl_ref[...], preferred_element_type=jnp.float32) * inv
  z = z + b_ref[...]
  o_ref[...] = jnp.where(z > 0, z, jnp.exp(jnp.minimum(z, 0.0)) - 1.0)


def _tc_layer2_body(h_ref, aggp_ref, degp_ref, wr_ref, wl_ref, b_ref, o_ref):
  agg = aggp_ref[0] + aggp_ref[1]
  deg = jnp.sum(degp_ref[...], axis=1, keepdims=True)
  inv = 1.0 / jnp.maximum(deg, 1.0)
  z = jnp.dot(h_ref[...], wr_ref[...], preferred_element_type=jnp.float32)
  z = z + jnp.dot(agg, wl_ref[...], preferred_element_type=jnp.float32) * inv
  o_ref[...] = z + b_ref[...]


def _tc_layer(body, feat, aggp, degp, w_root, w_rel, b, out_w):
  return pl.pallas_call(
      body,
      grid=(G,),
      in_specs=[
          pl.BlockSpec((NB, IN), lambda i: (i, 0)),
          pl.BlockSpec((NC, NB, H), lambda i: (0, i, 0)),
          pl.BlockSpec((NB, NW), lambda i: (i, 0)),
          pl.BlockSpec((IN, out_w), lambda i: (0, 0)),
          pl.BlockSpec((IN, out_w), lambda i: (0, 0)),
          pl.BlockSpec((1, out_w), lambda i: (0, 0)),
      ],
      out_specs=pl.BlockSpec((NB, out_w), lambda i: (i, 0)),
      out_shape=jax.ShapeDtypeStruct((N_PAD, out_w), jnp.float32),
  )(feat, aggp, degp, w_root, w_rel, b)


def kernel(x, edge_index, edge_type, W1_rel, W1_root, b1, W2_rel, W2_root, b2):
  del edge_type
  src = edge_index[0]
  dst = edge_index[1]
  x_pad = jnp.pad(x, ((0, N_PAD - N), (0, 0)))
  zeros_f = jnp.zeros((K, H), jnp.float32)
  zeros_deg = jnp.zeros((N_PAD,), jnp.float32)

  aggp1, degp = _sc_agg_deg(x_pad, src, dst, zeros_f, zeros_deg)
  aggp1 = aggp1.reshape(NC, N_PAD, H)
  degp = degp.T
  h = _tc_layer(_tc_layer1_body, x_pad, aggp1, degp, W1_root, W1_rel[0],
                b1.reshape(1, H), H)
  aggp2 = _sc_agg(h, src, dst, zeros_f, zeros_deg).reshape(NC, N_PAD, H)
  w2_root = jnp.pad(W2_root, ((0, 0), (0, H - OUT)))
  w2_rel = jnp.pad(W2_rel[0], ((0, 0), (0, H - OUT)))
  b2_p = jnp.pad(b2, (0, H - OUT)).reshape(1, H)
  out = _tc_layer(_tc_layer2_body, h, aggp2, degp, w2_root, w2_rel, b2_p, H)
  return out[:N, :OUT]


# speedup vs baseline: 10.2055x; 1.8037x over previous
import functools

import jax
import jax.numpy as jnp
from jax import lax
from jax.experimental import pallas as pl
from jax.experimental.pallas import tpu as pltpu
from jax.experimental.pallas import tpu_sc as plsc

N = 10000
E = 320000
IN = 128
H = 128
OUT = 2

NC = 2
NS = 16
NW = NC * NS
N_PAD = 10240
RPT = N_PAD // NS
EPW = E // NW
K = 80
NCHUNK = EPW // K
NB = 1024
G = N_PAD // NB


B = 1
NG = NCHUNK // B


def _make_sc_agg(with_deg):
  mesh = plsc.VectorSubcoreMesh(core_axis_name="c", subcore_axis_name="s")
  out_type = [jax.ShapeDtypeStruct((NC * N_PAD, H), jnp.float32)]
  if with_deg:
    out_type.append(jax.ShapeDtypeStruct((NW, N_PAD), jnp.float32))
  scratch = (
      [pltpu.VMEM((K,), jnp.int32)] * (2 * B)
      + [pltpu.VMEM((K,), jnp.int32)] * (2 * B)
      + [pltpu.VMEM((K, H), jnp.float32)] * (2 * B)
      + [pltpu.VMEM_SHARED((N_PAD, H), jnp.float32)]
  )
  if with_deg:
    scratch.append(pltpu.VMEM((N_PAD,), jnp.float32))
  scratch += [pltpu.SemaphoreType.DMA] * 3
  NSEG = RPT // K

  def body(feat, src, dst, zeros_f, zeros_deg, *refs):
    if with_deg:
      agg_out, deg_out = refs[0], refs[1]
      rest = refs[2:]
    else:
      agg_out = refs[0]
      rest = refs[1:]
    srci_flat = rest[:2 * B]
    dsti_flat = rest[2 * B:4 * B]
    rows_flat = rest[4 * B:6 * B]
    acc_s = rest[6 * B]
    if with_deg:
      deg_v = rest[6 * B + 1]
      gsem, ssem, isem = rest[6 * B + 2:6 * B + 5]
    else:
      gsem, ssem, isem = rest[6 * B + 1:6 * B + 4]
    srci = [srci_flat[bank * B:(bank + 1) * B] for bank in range(2)]
    dsti = [dsti_flat[bank * B:(bank + 1) * B] for bank in range(2)]
    rows = [rows_flat[bank * B:(bank + 1) * B] for bank in range(2)]
    c = lax.axis_index("c")
    s = lax.axis_index("s")
    w = c * NS + s
    row0 = s * RPT
    ebase = w * EPW
    stage = rows[0][0]

    pltpu.sync_copy(zeros_f, stage)
    for i in range(NSEG):
      pltpu.sync_copy(stage, acc_s.at[pl.ds(row0 + i * K, K)])
    if with_deg:
      pltpu.sync_copy(zeros_deg, deg_v)
    plsc.subcore_barrier()

    ones16 = jnp.full((16,), 1.0, jnp.float32)

    def fire_idx(g, bank):
      for b in range(B):
        off = ebase + g * (B * K) + b * K
        pltpu.async_copy(src.at[pl.ds(off, K)], srci[bank][b], isem)
        pltpu.async_copy(dst.at[pl.ds(off, K)], dsti[bank][b], isem)

    def drain_idx(bank):
      for b in range(B):
        pltpu.make_async_copy(src.at[pl.ds(0, K)], srci[bank][b], isem).wait()
        pltpu.make_async_copy(dst.at[pl.ds(0, K)], dsti[bank][b], isem).wait()

    def fire_gathers(bank):
      for b in range(B):
        pltpu.async_copy(feat.at[srci[bank][b]], rows[bank][b], gsem)

    def drain_gathers(bank):
      for b in range(B):
        pltpu.make_async_copy(feat.at[srci[bank][b]], rows[bank][b],
                              gsem).wait()

    def fire_scatters(bank):
      for b in range(B):
        pltpu.async_copy(rows[bank][b], acc_s.at[dsti[bank][b]],
                         ssem, add=True)

    def drain_scatters(bank):
      for b in range(B):
        pltpu.make_async_copy(rows[bank][b], acc_s.at[dsti[bank][b]],
                              ssem).wait()

    def deg_ops(bank):
      for b in range(B):
        for i in range(K // 16):
          dst16 = dsti[bank][b][pl.ds(i * 16, 16)]
          plsc.addupdate_scatter(deg_v, [dst16], ones16)

    def handle(g, cur, nxt):
      drain_gathers(cur)
      fire_scatters(cur)

      @pl.when(g < NG - 1)
      def _():
        drain_idx(nxt)
        fire_gathers(nxt)

      if with_deg:
        deg_ops(cur)
      drain_scatters(cur)

      @pl.when(g + 2 <= NG - 1)
      def _():
        fire_idx(g + 2, cur)

    for b in range(B):
      pltpu.sync_copy(src.at[pl.ds(ebase + b * K, K)], srci[0][b])
      pltpu.sync_copy(dst.at[pl.ds(ebase + b * K, K)], dsti[0][b])
    fire_gathers(0)
    fire_idx(1, 1)
    handle(0, 0, 1)

    def pair(p, carry):
      handle(2 * p + 1, 1, 0)
      handle(2 * p + 2, 0, 1)
      return carry

    lax.fori_loop(0, (NG - 1) // 2, pair, 0)
    plsc.subcore_barrier()

    out0 = c * N_PAD + row0
    for i in range(NSEG):
      pltpu.sync_copy(acc_s.at[pl.ds(row0 + i * K, K)], stage)
      pltpu.sync_copy(stage, agg_out.at[pl.ds(out0 + i * K, K)])
    if with_deg:
      pltpu.sync_copy(deg_v, deg_out.at[w])

  return pl.kernel(
      body, out_type=out_type, mesh=mesh, scratch_types=scratch,
      compiler_params=pltpu.CompilerParams(needs_layout_passes=False))


@functools.lru_cache(maxsize=None)
def _sc_agg_fn(with_deg):
  return _make_sc_agg(with_deg)


def _sc_agg_deg(*args):
  return _sc_agg_fn(True)(*args)


def _sc_agg(*args):
  return _sc_agg_fn(False)(*args)[0]


def _tc_layer1_body(x_ref, aggp_ref, degp_ref, wr_ref, wl_ref, b_ref, o_ref):
  agg = aggp_ref[0] + aggp_ref[1]
  deg = jnp.sum(degp_ref[...], axis=1, keepdims=True)
  inv = 1.0 / jnp.maximum(deg, 1.0)
  z = jnp.dot(x_ref[...], wr_ref[...], preferred_element_type=jnp.float32)
  z = z + jnp.dot(agg, wl_ref[...], preferred_element_type=jnp.float32) * inv
  z = z + b_ref[...]
  o_ref[...] = jnp.where(z > 0, z, jnp.exp(jnp.minimum(z, 0.0)) - 1.0)


def _tc_layer2_body(h_ref, aggp_ref, degp_ref, wr_ref, wl_ref, b_ref, o_ref):
  agg = aggp_ref[0] + aggp_ref[1]
  deg = jnp.sum(degp_ref[...], axis=1, keepdims=True)
  inv = 1.0 / jnp.maximum(deg, 1.0)
  z = jnp.dot(h_ref[...], wr_ref[...], preferred_element_type=jnp.float32)
  z = z + jnp.dot(agg, wl_ref[...], preferred_element_type=jnp.float32) * inv
  o_ref[...] = z + b_ref[...]


def _tc_layer(body, feat, aggp, degp, w_root, w_rel, b, out_w):
  return pl.pallas_call(
      body,
      grid=(G,),
      in_specs=[
          pl.BlockSpec((NB, IN), lambda i: (i, 0)),
          pl.BlockSpec((NC, NB, H), lambda i: (0, i, 0)),
          pl.BlockSpec((NB, NW), lambda i: (i, 0)),
          pl.BlockSpec((IN, out_w), lambda i: (0, 0)),
          pl.BlockSpec((IN, out_w), lambda i: (0, 0)),
          pl.BlockSpec((1, out_w), lambda i: (0, 0)),
      ],
      out_specs=pl.BlockSpec((NB, out_w), lambda i: (i, 0)),
      out_shape=jax.ShapeDtypeStruct((N_PAD, out_w), jnp.float32),
  )(feat, aggp, degp, w_root, w_rel, b)


def kernel(x, edge_index, edge_type, W1_rel, W1_root, b1, W2_rel, W2_root, b2):
  del edge_type
  src = edge_index[0]
  dst = edge_index[1]
  x_pad = jnp.pad(x, ((0, N_PAD - N), (0, 0)))
  zeros_f = jnp.zeros((K, H), jnp.float32)
  zeros_deg = jnp.zeros((N_PAD,), jnp.float32)

  aggp1, degp = _sc_agg_deg(x_pad, src, dst, zeros_f, zeros_deg)
  aggp1 = aggp1.reshape(NC, N_PAD, H)
  degp = degp.T
  h = _tc_layer(_tc_layer1_body, x_pad, aggp1, degp, W1_root, W1_rel[0],
                b1.reshape(1, H), H)
  aggp2 = _sc_agg(h, src, dst, zeros_f, zeros_deg).reshape(NC, N_PAD, H)
  w2_root = jnp.pad(W2_root, ((0, 0), (0, H - OUT)))
  w2_rel = jnp.pad(W2_rel[0], ((0, 0), (0, H - OUT)))
  b2_p = jnp.pad(b2, (0, H - OUT)).reshape(1, H)
  out = _tc_layer(_tc_layer2_body, h, aggp2, degp, w2_root, w2_rel, b2_p, H)
  return out[:N, :OUT]


# speedup vs baseline: 10.3117x; 1.0104x over previous
import functools

import jax
import jax.numpy as jnp
from jax import lax
from jax.experimental import pallas as pl
from jax.experimental.pallas import tpu as pltpu
from jax.experimental.pallas import tpu_sc as plsc

N = 10000
E = 320000
IN = 128
H = 128
OUT = 2

NC = 2
NS = 16
NW = NC * NS
N_PAD = 10240
RPT = N_PAD // NS
EPW = E // NW
K = 40
NCHUNK = EPW // K
NB = 1024
G = N_PAD // NB


B = 2
NG = NCHUNK // B


def _make_sc_agg(with_deg):
  mesh = plsc.VectorSubcoreMesh(core_axis_name="c", subcore_axis_name="s")
  out_type = [jax.ShapeDtypeStruct((NC * N_PAD, H), jnp.float32)]
  if with_deg:
    out_type.append(jax.ShapeDtypeStruct((NW, N_PAD), jnp.float32))
  scratch = (
      [pltpu.VMEM((K,), jnp.int32)] * (2 * B)
      + [pltpu.VMEM((K,), jnp.int32)] * (2 * B)
      + [pltpu.VMEM((K, H), jnp.float32)] * (2 * B)
      + [pltpu.VMEM_SHARED((N_PAD, H), jnp.float32)]
  )
  if with_deg:
    scratch.append(pltpu.VMEM((N_PAD,), jnp.float32))
  scratch += [pltpu.SemaphoreType.DMA] * 3
  NSEG = RPT // K

  def body(feat, src, dst, zeros_f, zeros_deg, *refs):
    if with_deg:
      agg_out, deg_out = refs[0], refs[1]
      rest = refs[2:]
    else:
      agg_out = refs[0]
      rest = refs[1:]
    srci_flat = rest[:2 * B]
    dsti_flat = rest[2 * B:4 * B]
    rows_flat = rest[4 * B:6 * B]
    acc_s = rest[6 * B]
    if with_deg:
      deg_v = rest[6 * B + 1]
      gsem, ssem, isem = rest[6 * B + 2:6 * B + 5]
    else:
      gsem, ssem, isem = rest[6 * B + 1:6 * B + 4]
    srci = [srci_flat[bank * B:(bank + 1) * B] for bank in range(2)]
    dsti = [dsti_flat[bank * B:(bank + 1) * B] for bank in range(2)]
    rows = [rows_flat[bank * B:(bank + 1) * B] for bank in range(2)]
    c = lax.axis_index("c")
    s = lax.axis_index("s")
    w = c * NS + s
    row0 = s * RPT
    ebase = w * EPW
    stage = rows[0][0]

    pltpu.sync_copy(zeros_f, stage)
    for i in range(NSEG):
      pltpu.async_copy(stage, acc_s.at[pl.ds(row0 + i * K, K)], gsem)
    if with_deg:
      pltpu.sync_copy(zeros_deg, deg_v)
    for i in range(NSEG):
      pltpu.make_async_copy(stage, acc_s.at[pl.ds(row0, K)], gsem).wait()
    plsc.subcore_barrier()

    ones16 = jnp.full((16,), 1.0, jnp.float32)

    def fire_idx(g, bank):
      for b in range(B):
        off = ebase + g * (B * K) + b * K
        pltpu.async_copy(src.at[pl.ds(off, K)], srci[bank][b], isem)
        pltpu.async_copy(dst.at[pl.ds(off, K)], dsti[bank][b], isem)

    def drain_idx(bank):
      for b in range(B):
        pltpu.make_async_copy(src.at[pl.ds(0, K)], srci[bank][b], isem).wait()
        pltpu.make_async_copy(dst.at[pl.ds(0, K)], dsti[bank][b], isem).wait()

    def fire_gathers(bank):
      for b in range(B):
        pltpu.async_copy(feat.at[srci[bank][b]], rows[bank][b], gsem)

    def drain_gathers(bank):
      for b in range(B):
        pltpu.make_async_copy(feat.at[srci[bank][b]], rows[bank][b],
                              gsem).wait()

    def fire_scatters(bank):
      for b in range(B):
        pltpu.async_copy(rows[bank][b], acc_s.at[dsti[bank][b]],
                         ssem, add=True)

    def drain_scatters(bank):
      for b in range(B):
        pltpu.make_async_copy(rows[bank][b], acc_s.at[dsti[bank][b]],
                              ssem).wait()

    tail = K % 16
    tail_mask = (lax.iota(jnp.int32, 16) >= 16 - tail) if tail else None

    def deg_ops(bank):
      for b in range(B):
        for i in range(K // 16):
          dst16 = dsti[bank][b][pl.ds(i * 16, 16)]
          plsc.addupdate_scatter(deg_v, [dst16], ones16)
        if tail:
          dst16 = dsti[bank][b][pl.ds(K - 16, 16)]
          plsc.addupdate_scatter(deg_v, [dst16], ones16, mask=tail_mask)

    def handle(g, cur, nxt):
      drain_gathers(cur)
      fire_scatters(cur)

      @pl.when(g < NG - 1)
      def _():
        drain_idx(nxt)
        fire_gathers(nxt)

      if with_deg:
        deg_ops(cur)
      drain_scatters(cur)

      @pl.when(g + 2 <= NG - 1)
      def _():
        fire_idx(g + 2, cur)

    for b in range(B):
      pltpu.sync_copy(src.at[pl.ds(ebase + b * K, K)], srci[0][b])
      pltpu.sync_copy(dst.at[pl.ds(ebase + b * K, K)], dsti[0][b])
    fire_gathers(0)
    fire_idx(1, 1)
    handle(0, 0, 1)

    def pair(p, carry):
      handle(2 * p + 1, 1, 0)
      handle(2 * p + 2, 0, 1)
      return carry

    lax.fori_loop(0, (NG - 1) // 2, pair, 0)
    plsc.subcore_barrier()

    out0 = c * N_PAD + row0
    bufs = rows[0] + rows[1]
    nring = len(bufs)
    if with_deg:
      pltpu.async_copy(deg_v, deg_out.at[w], isem)
    for i in range(NSEG):
      b = bufs[i % nring]
      if i >= nring:
        pltpu.make_async_copy(b, agg_out.at[pl.ds(out0, K)], ssem).wait()
      pltpu.sync_copy(acc_s.at[pl.ds(row0 + i * K, K)], b)
      pltpu.async_copy(b, agg_out.at[pl.ds(out0 + i * K, K)], ssem)
    for i in range(max(NSEG - nring, 0), NSEG):
      pltpu.make_async_copy(bufs[i % nring], agg_out.at[pl.ds(out0, K)],
                            ssem).wait()
    if with_deg:
      pltpu.make_async_copy(deg_v, deg_out.at[w], isem).wait()

  return pl.kernel(
      body, out_type=out_type, mesh=mesh, scratch_types=scratch,
      compiler_params=pltpu.CompilerParams(needs_layout_passes=False))


@functools.lru_cache(maxsize=None)
def _sc_agg_fn(with_deg):
  return _make_sc_agg(with_deg)


def _sc_agg_deg(*args):
  return _sc_agg_fn(True)(*args)


def _sc_agg(*args):
  return _sc_agg_fn(False)(*args)[0]


def _tc_layer1_body(x_ref, aggp_ref, degp_ref, wr_ref, wl_ref, b_ref, o_ref):
  agg = aggp_ref[0] + aggp_ref[1]
  deg = jnp.sum(degp_ref[...], axis=1, keepdims=True)
  inv = 1.0 / jnp.maximum(deg, 1.0)
  z = jnp.dot(x_ref[...], wr_ref[...], preferred_element_type=jnp.float32)
  z = z + jnp.dot(agg, wl_ref[...], preferred_element_type=jnp.float32) * inv
  z = z + b_ref[...]
  o_ref[...] = jnp.where(z > 0, z, jnp.exp(jnp.minimum(z, 0.0)) - 1.0)


def _tc_layer2_body(h_ref, aggp_ref, degp_ref, wr_ref, wl_ref, b_ref, o_ref):
  agg = aggp_ref[0] + aggp_ref[1]
  deg = jnp.sum(degp_ref[...], axis=1, keepdims=True)
  inv = 1.0 / jnp.maximum(deg, 1.0)
  z = jnp.dot(h_ref[...], wr_ref[...], preferred_element_type=jnp.float32)
  z = z + jnp.dot(agg, wl_ref[...], preferred_element_type=jnp.float32) * inv
  o_ref[...] = z + b_ref[...]


def _tc_layer(body, feat, aggp, degp, w_root, w_rel, b, out_w):
  return pl.pallas_call(
      body,
      grid=(G,),
      in_specs=[
          pl.BlockSpec((NB, IN), lambda i: (i, 0)),
          pl.BlockSpec((NC, NB, H), lambda i: (0, i, 0)),
          pl.BlockSpec((NB, NW), lambda i: (i, 0)),
          pl.BlockSpec((IN, out_w), lambda i: (0, 0)),
          pl.BlockSpec((IN, out_w), lambda i: (0, 0)),
          pl.BlockSpec((1, out_w), lambda i: (0, 0)),
      ],
      out_specs=pl.BlockSpec((NB, out_w), lambda i: (i, 0)),
      out_shape=jax.ShapeDtypeStruct((N_PAD, out_w), jnp.float32),
  )(feat, aggp, degp, w_root, w_rel, b)


def kernel(x, edge_index, edge_type, W1_rel, W1_root, b1, W2_rel, W2_root, b2):
  del edge_type
  src = edge_index[0]
  dst = edge_index[1]
  x_pad = jnp.pad(x, ((0, N_PAD - N), (0, 0)))
  zeros_f = jnp.zeros((K, H), jnp.float32)
  zeros_deg = jnp.zeros((N_PAD,), jnp.float32)

  aggp1, degp = _sc_agg_deg(x_pad, src, dst, zeros_f, zeros_deg)
  aggp1 = aggp1.reshape(NC, N_PAD, H)
  degp = degp.T
  h = _tc_layer(_tc_layer1_body, x_pad, aggp1, degp, W1_root, W1_rel[0],
                b1.reshape(1, H), H)
  aggp2 = _sc_agg(h, src, dst, zeros_f, zeros_deg).reshape(NC, N_PAD, H)
  w2_root = jnp.pad(W2_root, ((0, 0), (0, H - OUT)))
  w2_rel = jnp.pad(W2_rel[0], ((0, 0), (0, H - OUT)))
  b2_p = jnp.pad(b2, (0, H - OUT)).reshape(1, H)
  out = _tc_layer(_tc_layer2_body, h, aggp2, degp, w2_root, w2_rel, b2_p, H)
  return out[:N, :OUT]


# speedup vs baseline: 10.3684x; 1.0055x over previous
import functools

import jax
import jax.numpy as jnp
from jax import lax
from jax.experimental import pallas as pl
from jax.experimental.pallas import tpu as pltpu
from jax.experimental.pallas import tpu_sc as plsc

N = 10000
E = 320000
IN = 128
H = 128
OUT = 2

NC = 2
NS = 16
NW = NC * NS
N_PAD = 10240
RPT = N_PAD // NS
EPW = E // NW
K = 40
NCHUNK = EPW // K
NB = 1024
G = N_PAD // NB


B = 2
NG = NCHUNK // B


def _make_sc_agg(with_deg):
  mesh = plsc.VectorSubcoreMesh(core_axis_name="c", subcore_axis_name="s")
  out_type = [jax.ShapeDtypeStruct((NC * N_PAD, H), jnp.float32)]
  if with_deg:
    out_type.append(jax.ShapeDtypeStruct((NW, N_PAD), jnp.float32))
  scratch = (
      [pltpu.VMEM((K,), jnp.int32)] * (2 * B)
      + [pltpu.VMEM((K,), jnp.int32)] * (2 * B)
      + [pltpu.VMEM((K, H), jnp.float32)] * (2 * B)
      + [pltpu.VMEM_SHARED((N_PAD, H), jnp.float32)]
  )
  if with_deg:
    scratch.append(pltpu.VMEM((N_PAD,), jnp.float32))
  scratch += [pltpu.SemaphoreType.DMA] * 3
  NSEG = RPT // K

  def body(feat, src, dst, zeros_f, zeros_deg, *refs):
    if with_deg:
      agg_out, deg_out = refs[0], refs[1]
      rest = refs[2:]
    else:
      agg_out = refs[0]
      rest = refs[1:]
    srci_flat = rest[:2 * B]
    dsti_flat = rest[2 * B:4 * B]
    rows_flat = rest[4 * B:6 * B]
    acc_s = rest[6 * B]
    if with_deg:
      deg_v = rest[6 * B + 1]
      gsem, ssem, isem = rest[6 * B + 2:6 * B + 5]
    else:
      gsem, ssem, isem = rest[6 * B + 1:6 * B + 4]
    srci = [srci_flat[bank * B:(bank + 1) * B] for bank in range(2)]
    dsti = [dsti_flat[bank * B:(bank + 1) * B] for bank in range(2)]
    rows = [rows_flat[bank * B:(bank + 1) * B] for bank in range(2)]
    c = lax.axis_index("c")
    s = lax.axis_index("s")
    w = c * NS + s
    row0 = s * RPT
    ebase = w * EPW
    stage = rows[0][0]

    pltpu.sync_copy(zeros_f, stage)
    for i in range(NSEG):
      pltpu.async_copy(stage, acc_s.at[pl.ds(row0 + i * K, K)], gsem)
    if with_deg:
      pltpu.sync_copy(zeros_deg, deg_v)
    for i in range(NSEG):
      pltpu.make_async_copy(stage, acc_s.at[pl.ds(row0, K)], gsem).wait()
    plsc.subcore_barrier()

    ones16 = jnp.full((16,), 1.0, jnp.float32)

    def fire_idx(g, bank):
      for b in range(B):
        off = ebase + g * (B * K) + b * K
        pltpu.async_copy(src.at[pl.ds(off, K)], srci[bank][b], isem)
        pltpu.async_copy(dst.at[pl.ds(off, K)], dsti[bank][b], isem)

    def drain_idx(bank):
      for b in range(B):
        pltpu.make_async_copy(src.at[pl.ds(0, K)], srci[bank][b], isem).wait()
        pltpu.make_async_copy(dst.at[pl.ds(0, K)], dsti[bank][b], isem).wait()

    def fire_gathers(bank):
      for b in range(B):
        pltpu.async_copy(feat.at[srci[bank][b]], rows[bank][b], gsem)

    def drain_gathers(bank):
      for b in range(B):
        pltpu.make_async_copy(feat.at[srci[bank][b]], rows[bank][b],
                              gsem).wait()

    def fire_scatters(bank):
      for b in range(B):
        pltpu.async_copy(rows[bank][b], acc_s.at[dsti[bank][b]],
                         ssem, add=True)

    def drain_scatters(bank):
      for b in range(B):
        pltpu.make_async_copy(rows[bank][b], acc_s.at[dsti[bank][b]],
                              ssem).wait()

    tail = K % 16
    tail_mask = (lax.iota(jnp.int32, 16) >= 16 - tail) if tail else None

    def deg_ops(bank):
      for b in range(B):
        for i in range(K // 16):
          dst16 = dsti[bank][b][pl.ds(i * 16, 16)]
          plsc.addupdate_scatter(deg_v, [dst16], ones16)
        if tail:
          dst16 = dsti[bank][b][pl.ds(K - 16, 16)]
          plsc.addupdate_scatter(deg_v, [dst16], ones16, mask=tail_mask)

    def handle(g, cur, nxt):
      drain_gathers(cur)
      fire_scatters(cur)

      @pl.when(g < NG - 1)
      def _():
        drain_idx(nxt)
        fire_gathers(nxt)

      if with_deg:
        deg_ops(cur)
      drain_scatters(cur)

      @pl.when(g + 2 <= NG - 1)
      def _():
        fire_idx(g + 2, cur)

    for b in range(B):
      pltpu.sync_copy(src.at[pl.ds(ebase + b * K, K)], srci[0][b])
      pltpu.sync_copy(dst.at[pl.ds(ebase + b * K, K)], dsti[0][b])
    fire_gathers(0)
    fire_idx(1, 1)
    handle(0, 0, 1)

    def pair(p, carry):
      handle(2 * p + 1, 1, 0)
      handle(2 * p + 2, 0, 1)
      return carry

    lax.fori_loop(0, (NG - 1) // 2, pair, 0)
    plsc.subcore_barrier()

    out0 = c * N_PAD + row0
    bufs = rows[0] + rows[1]
    nring = len(bufs)
    if with_deg:
      pltpu.async_copy(deg_v, deg_out.at[w], isem)
    for i in range(NSEG):
      b = bufs[i % nring]
      if i >= nring:
        pltpu.make_async_copy(b, agg_out.at[pl.ds(out0, K)], ssem).wait()
      pltpu.sync_copy(acc_s.at[pl.ds(row0 + i * K, K)], b)
      pltpu.async_copy(b, agg_out.at[pl.ds(out0 + i * K, K)], ssem)
    for i in range(max(NSEG - nring, 0), NSEG):
      pltpu.make_async_copy(bufs[i % nring], agg_out.at[pl.ds(out0, K)],
                            ssem).wait()
    if with_deg:
      pltpu.make_async_copy(deg_v, deg_out.at[w], isem).wait()

  return pl.kernel(
      body, out_type=out_type, mesh=mesh, scratch_types=scratch,
      compiler_params=pltpu.CompilerParams(needs_layout_passes=False))


@functools.lru_cache(maxsize=None)
def _sc_agg_fn(with_deg):
  return _make_sc_agg(with_deg)


def _sc_agg_deg(*args):
  return _sc_agg_fn(True)(*args)


def _sc_agg(*args):
  return _sc_agg_fn(False)(*args)[0]


def _tc_root_body(x_ref, wr_ref, b_ref, o_ref):
  o_ref[...] = (jnp.dot(x_ref[...], wr_ref[...],
                        preferred_element_type=jnp.float32) + b_ref[...])


def _tc_root(feat, w_root, b, out_w):
  n = feat.shape[0]
  return pl.pallas_call(
      _tc_root_body,
      grid=(G,),
      in_specs=[
          pl.BlockSpec((NB, IN), lambda i: (i, 0)),
          pl.BlockSpec((IN, out_w), lambda i: (0, 0)),
          pl.BlockSpec((1, out_w), lambda i: (0, 0)),
      ],
      out_specs=pl.BlockSpec((NB, out_w), lambda i: (i, 0)),
      out_shape=jax.ShapeDtypeStruct((N_PAD, out_w), jnp.float32),
  )(feat, w_root, b)


def _tc_fin_body(elu, root_ref, aggp_ref, degp_ref, wl_ref, o_ref):
  agg = aggp_ref[0] + aggp_ref[1]
  deg = jnp.sum(degp_ref[...], axis=1, keepdims=True)
  inv = 1.0 / jnp.maximum(deg, 1.0)
  z = root_ref[...] + jnp.dot(agg, wl_ref[...],
                              preferred_element_type=jnp.float32) * inv
  if elu:
    z = jnp.where(z > 0, z, jnp.exp(jnp.minimum(z, 0.0)) - 1.0)
  o_ref[...] = z


def _tc_fin(elu, root, aggp, degp, w_rel, out_w):
  return pl.pallas_call(
      functools.partial(_tc_fin_body, elu),
      grid=(G,),
      in_specs=[
          pl.BlockSpec((NB, out_w), lambda i: (i, 0)),
          pl.BlockSpec((NC, NB, H), lambda i: (0, i, 0)),
          pl.BlockSpec((NB, NW), lambda i: (i, 0)),
          pl.BlockSpec((H, out_w), lambda i: (0, 0)),
      ],
      out_specs=pl.BlockSpec((NB, out_w), lambda i: (i, 0)),
      out_shape=jax.ShapeDtypeStruct((N_PAD, out_w), jnp.float32),
  )(root, aggp, degp, w_rel)


OUTW = 8


def kernel(x, edge_index, edge_type, W1_rel, W1_root, b1, W2_rel, W2_root, b2):
  del edge_type
  src = edge_index[0]
  dst = edge_index[1]
  zeros_f = jnp.zeros((K, H), jnp.float32)
  zeros_deg = jnp.zeros((N_PAD,), jnp.float32)

  aggp1, degp = _sc_agg_deg(x, src, dst, zeros_f, zeros_deg)
  root1 = _tc_root(x, W1_root, b1.reshape(1, H), H)
  aggp1 = aggp1.reshape(NC, N_PAD, H)
  degp = degp.T
  h = _tc_fin(True, root1, aggp1, degp, W1_rel[0], H)

  aggp2 = _sc_agg(h, src, dst, zeros_f, zeros_deg).reshape(NC, N_PAD, H)
  w2_root = jnp.pad(W2_root, ((0, 0), (0, OUTW - OUT)))
  w2_rel = jnp.pad(W2_rel[0], ((0, 0), (0, OUTW - OUT)))
  b2_p = jnp.pad(b2, (0, OUTW - OUT)).reshape(1, OUTW)
  root2 = _tc_root(h, w2_root, b2_p, OUTW)
  out = _tc_fin(False, root2, aggp2, degp, w2_rel, OUTW)
  return out[:N, :OUT]


# speedup vs baseline: 12.4107x; 1.1970x over previous
import functools

import jax
import jax.numpy as jnp
from jax import lax
from jax.experimental import pallas as pl
from jax.experimental.pallas import tpu as pltpu
from jax.experimental.pallas import tpu_sc as plsc

N = 10000
E = 320000
IN = 128
H = 128
OUT = 2

NC = 2
NS = 16
NW = NC * NS
N_PAD = 10240
RPT = N_PAD // NS
EPW = E // NW
K = 40
NCHUNK = EPW // K
NB = 1024
G = N_PAD // NB


RING = 5
NSTEP = NCHUNK // RING


def _make_sc_agg(with_deg):
  mesh = plsc.VectorSubcoreMesh(core_axis_name="c", subcore_axis_name="s")
  out_type = [jax.ShapeDtypeStruct((NC * N_PAD, H), jnp.float32)]
  if with_deg:
    out_type.append(jax.ShapeDtypeStruct((NW, N_PAD), jnp.float32))
  scratch = (
      [pltpu.VMEM((K,), jnp.int32)] * (2 * RING)
      + [pltpu.VMEM((K,), jnp.int32)] * (2 * RING)
      + [pltpu.VMEM((K, H), jnp.float32)] * RING
      + [pltpu.VMEM_SHARED((N_PAD, H), jnp.float32)]
  )
  if with_deg:
    scratch.append(pltpu.VMEM((N_PAD,), jnp.float32))
  scratch += [pltpu.SemaphoreType.DMA] * 3
  NSEG = RPT // K

  def body(feat, src, dst, zeros_f, zeros_deg, *refs):
    if with_deg:
      agg_out, deg_out = refs[0], refs[1]
      rest = refs[2:]
    else:
      agg_out = refs[0]
      rest = refs[1:]
    srci_flat = rest[:2 * RING]
    dsti_flat = rest[2 * RING:4 * RING]
    rows = rest[4 * RING:5 * RING]
    acc_s = rest[5 * RING]
    if with_deg:
      deg_v = rest[5 * RING + 1]
      gsem, ssem, isem = rest[5 * RING + 2:5 * RING + 5]
    else:
      gsem, ssem, isem = rest[5 * RING + 1:5 * RING + 4]
    srci = [srci_flat[bank * RING:(bank + 1) * RING] for bank in range(2)]
    dsti = [dsti_flat[bank * RING:(bank + 1) * RING] for bank in range(2)]
    c = lax.axis_index("c")
    s = lax.axis_index("s")
    w = c * NS + s
    row0 = s * RPT
    ebase = w * EPW
    stage = rows[0]

    pltpu.sync_copy(zeros_f, stage)
    for i in range(NSEG):
      pltpu.async_copy(stage, acc_s.at[pl.ds(row0 + i * K, K)], gsem)
    if with_deg:
      pltpu.sync_copy(zeros_deg, deg_v)
    for i in range(NSEG):
      pltpu.make_async_copy(stage, acc_s.at[pl.ds(row0, K)], gsem).wait()
    plsc.subcore_barrier()

    ones16 = jnp.full((16,), 1.0, jnp.float32)
    tail = K % 16
    tail_mask = (lax.iota(jnp.int32, 16) >= 16 - tail) if tail else None

    def fire_idx(t, bank):
      for b in range(RING):
        off = ebase + t * (RING * K) + b * K
        pltpu.async_copy(src.at[pl.ds(off, K)], srci[bank][b], isem)
        pltpu.async_copy(dst.at[pl.ds(off, K)], dsti[bank][b], isem)

    def drain_idx(bank):
      for b in range(RING):
        pltpu.make_async_copy(src.at[pl.ds(0, K)], srci[bank][b], isem).wait()
        pltpu.make_async_copy(dst.at[pl.ds(0, K)], dsti[bank][b], isem).wait()

    def drain_scatters(bank):
      for b in range(RING):
        pltpu.make_async_copy(rows[b], acc_s.at[dsti[bank][b]], ssem).wait()

    def deg_ops(bank):
      for b in range(RING):
        for i in range(K // 16):
          dst16 = dsti[bank][b][pl.ds(i * 16, 16)]
          plsc.addupdate_scatter(deg_v, [dst16], ones16)
        if tail:
          dst16 = dsti[bank][b][pl.ds(K - 16, 16)]
          plsc.addupdate_scatter(deg_v, [dst16], ones16, mask=tail_mask)

    def step(t, bank, prev_bank):
      @pl.when(t > 0)
      def _():
        drain_scatters(prev_bank)
      drain_idx(bank)
      for b in range(RING):
        pltpu.async_copy(feat.at[srci[bank][b]], rows[b], gsem)

      @pl.when(t + 1 < NSTEP)
      def _():
        fire_idx(t + 1, 1 - bank)

      for b in range(RING):
        pltpu.make_async_copy(feat.at[srci[bank][b]], rows[b], gsem).wait()
        pltpu.async_copy(rows[b], acc_s.at[dsti[bank][b]], ssem, add=True)
      if with_deg:
        deg_ops(bank)

    fire_idx(0, 0)

    def pair(p, carry):
      step(2 * p, 0, 1)
      step(2 * p + 1, 1, 0)
      return carry

    lax.fori_loop(0, NSTEP // 2, pair, 0)
    drain_scatters(1)
    plsc.subcore_barrier()

    out0 = c * N_PAD + row0
    bufs = list(rows)
    nring = len(bufs)
    if with_deg:
      pltpu.async_copy(deg_v, deg_out.at[w], isem)
    for i in range(NSEG):
      b = bufs[i % nring]
      if i >= nring:
        pltpu.make_async_copy(b, agg_out.at[pl.ds(out0, K)], ssem).wait()
      pltpu.sync_copy(acc_s.at[pl.ds(row0 + i * K, K)], b)
      pltpu.async_copy(b, agg_out.at[pl.ds(out0 + i * K, K)], ssem)
    for i in range(max(NSEG - nring, 0), NSEG):
      pltpu.make_async_copy(bufs[i % nring], agg_out.at[pl.ds(out0, K)],
                            ssem).wait()
    if with_deg:
      pltpu.make_async_copy(deg_v, deg_out.at[w], isem).wait()

  return pl.kernel(
      body, out_type=out_type, mesh=mesh, scratch_types=scratch,
      compiler_params=pltpu.CompilerParams(needs_layout_passes=False))


@functools.lru_cache(maxsize=None)
def _sc_agg_fn(with_deg):
  return _make_sc_agg(with_deg)


def _sc_agg_deg(*args):
  return _sc_agg_fn(True)(*args)


def _sc_agg(*args):
  return _sc_agg_fn(False)(*args)[0]


def _tc_root_body(x_ref, wr_ref, b_ref, o_ref):
  o_ref[...] = (jnp.dot(x_ref[...], wr_ref[...],
                        preferred_element_type=jnp.float32) + b_ref[...])


def _tc_root(feat, w_root, b, out_w):
  n = feat.shape[0]
  return pl.pallas_call(
      _tc_root_body,
      grid=(G,),
      in_specs=[
          pl.BlockSpec((NB, IN), lambda i: (i, 0)),
          pl.BlockSpec((IN, out_w), lambda i: (0, 0)),
          pl.BlockSpec((1, out_w), lambda i: (0, 0)),
      ],
      out_specs=pl.BlockSpec((NB, out_w), lambda i: (i, 0)),
      out_shape=jax.ShapeDtypeStruct((N_PAD, out_w), jnp.float32),
  )(feat, w_root, b)


def _tc_fin_body(elu, root_ref, aggp_ref, degp_ref, wl_ref, o_ref):
  agg = aggp_ref[0] + aggp_ref[1]
  deg = jnp.sum(degp_ref[...], axis=1, keepdims=True)
  inv = 1.0 / jnp.maximum(deg, 1.0)
  z = root_ref[...] + jnp.dot(agg, wl_ref[...],
                              preferred_element_type=jnp.float32) * inv
  if elu:
    z = jnp.where(z > 0, z, jnp.exp(jnp.minimum(z, 0.0)) - 1.0)
  o_ref[...] = z


def _tc_fin(elu, root, aggp, degp, w_rel, out_w):
  return pl.pallas_call(
      functools.partial(_tc_fin_body, elu),
      grid=(G,),
      in_specs=[
          pl.BlockSpec((NB, out_w), lambda i: (i, 0)),
          pl.BlockSpec((NC, NB, H), lambda i: (0, i, 0)),
          pl.BlockSpec((NB, NW), lambda i: (i, 0)),
          pl.BlockSpec((H, out_w), lambda i: (0, 0)),
      ],
      out_specs=pl.BlockSpec((NB, out_w), lambda i: (i, 0)),
      out_shape=jax.ShapeDtypeStruct((N_PAD, out_w), jnp.float32),
  )(root, aggp, degp, w_rel)


OUTW = 8


def kernel(x, edge_index, edge_type, W1_rel, W1_root, b1, W2_rel, W2_root, b2):
  del edge_type
  src = edge_index[0]
  dst = edge_index[1]
  zeros_f = jnp.zeros((K, H), jnp.float32)
  zeros_deg = jnp.zeros((N_PAD,), jnp.float32)

  aggp1, degp = _sc_agg_deg(x, src, dst, zeros_f, zeros_deg)
  root1 = _tc_root(x, W1_root, b1.reshape(1, H), H)
  aggp1 = aggp1.reshape(NC, N_PAD, H)
  degp = degp.T
  h = _tc_fin(True, root1, aggp1, degp, W1_rel[0], H)

  aggp2 = _sc_agg(h, src, dst, zeros_f, zeros_deg).reshape(NC, N_PAD, H)
  w2_root = jnp.pad(W2_root, ((0, 0), (0, OUTW - OUT)))
  w2_rel = jnp.pad(W2_rel[0], ((0, 0), (0, OUTW - OUT)))
  b2_p = jnp.pad(b2, (0, OUTW - OUT)).reshape(1, OUTW)
  root2 = _tc_root(h, w2_root, b2_p, OUTW)
  out = _tc_fin(False, root2, aggp2, degp, w2_rel, OUTW)
  return out[:N, :OUT]


# speedup vs baseline: 13.2613x; 1.0685x over previous
import functools

import jax
import jax.numpy as jnp
from jax import lax
from jax.experimental import pallas as pl
from jax.experimental.pallas import tpu as pltpu
from jax.experimental.pallas import tpu_sc as plsc

N = 10000
E = 320000
IN = 128
H = 128
OUT = 2

NC = 2
NS = 16
NW = NC * NS
N_PAD = 10240
RPT = N_PAD // NS
EPW = E // NW
K = 40
NCHUNK = EPW // K
NB = 1024
G = N_PAD // NB


RING = 5
NSTEP = NCHUNK // RING


def _make_sc_agg(with_deg):
  mesh = plsc.VectorSubcoreMesh(core_axis_name="c", subcore_axis_name="s")
  out_type = [jax.ShapeDtypeStruct((NC * N_PAD, H), jnp.float32)]
  if with_deg:
    out_type.append(jax.ShapeDtypeStruct((NW, N_PAD), jnp.float32))
  scratch = (
      [pltpu.VMEM((K,), jnp.int32)] * (2 * RING)
      + [pltpu.VMEM((K,), jnp.int32)] * (2 * RING)
      + [pltpu.VMEM((K, H), jnp.float32)] * RING
      + [pltpu.VMEM_SHARED((N_PAD, H), jnp.float32)]
  )
  if with_deg:
    scratch.append(pltpu.VMEM((N_PAD,), jnp.float32))
  scratch += [pltpu.SemaphoreType.DMA] * (3 + RING)
  NSEG = RPT // K

  def body(feat, src, dst, zeros_f, zeros_deg, *refs):
    if with_deg:
      agg_out, deg_out = refs[0], refs[1]
      rest = refs[2:]
    else:
      agg_out = refs[0]
      rest = refs[1:]
    srci_flat = rest[:2 * RING]
    dsti_flat = rest[2 * RING:4 * RING]
    rows = rest[4 * RING:5 * RING]
    acc_s = rest[5 * RING]
    base = 5 * RING + (2 if with_deg else 1)
    if with_deg:
      deg_v = rest[5 * RING + 1]
    gsem, ssem, isem = rest[base:base + 3]
    ssems = rest[base + 3:base + 3 + RING]
    srci = [srci_flat[bank * RING:(bank + 1) * RING] for bank in range(2)]
    dsti = [dsti_flat[bank * RING:(bank + 1) * RING] for bank in range(2)]
    c = lax.axis_index("c")
    s = lax.axis_index("s")
    w = c * NS + s
    row0 = s * RPT
    ebase = w * EPW
    stage = rows[0]

    pltpu.sync_copy(zeros_f, stage)
    for i in range(NSEG):
      pltpu.async_copy(stage, acc_s.at[pl.ds(row0 + i * K, K)], gsem)
    if with_deg:
      pltpu.sync_copy(zeros_deg, deg_v)
    for i in range(NSEG):
      pltpu.make_async_copy(stage, acc_s.at[pl.ds(row0, K)], gsem).wait()
    plsc.subcore_barrier()

    ones16 = jnp.full((16,), 1.0, jnp.float32)
    tail = K % 16
    tail_mask = (lax.iota(jnp.int32, 16) >= 16 - tail) if tail else None

    def fire_idx(t, bank):
      for b in range(RING):
        off = ebase + t * (RING * K) + b * K
        pltpu.async_copy(src.at[pl.ds(off, K)], srci[bank][b], isem)
        pltpu.async_copy(dst.at[pl.ds(off, K)], dsti[bank][b], isem)

    def drain_idx(bank):
      for b in range(RING):
        pltpu.make_async_copy(src.at[pl.ds(0, K)], srci[bank][b], isem).wait()
        pltpu.make_async_copy(dst.at[pl.ds(0, K)], dsti[bank][b], isem).wait()

    def drain_scatters(bank):
      for b in range(RING):
        pltpu.make_async_copy(rows[b], acc_s.at[dsti[bank][b]],
                              ssems[b]).wait()

    def deg_ops(bank):
      for b in range(RING):
        for i in range(K // 16):
          dst16 = dsti[bank][b][pl.ds(i * 16, 16)]
          plsc.addupdate_scatter(deg_v, [dst16], ones16)
        if tail:
          dst16 = dsti[bank][b][pl.ds(K - 16, 16)]
          plsc.addupdate_scatter(deg_v, [dst16], ones16, mask=tail_mask)

    def step(t, bank, prev_bank):
      drain_idx(bank)
      for b in range(RING):
        @pl.when(t > 0)
        def _():
          pltpu.make_async_copy(rows[b], acc_s.at[dsti[prev_bank][b]],
                                ssems[b]).wait()
        pltpu.async_copy(feat.at[srci[bank][b]], rows[b], gsem)

      @pl.when(t + 1 < NSTEP)
      def _():
        fire_idx(t + 1, 1 - bank)

      for b in range(RING):
        pltpu.make_async_copy(feat.at[srci[bank][b]], rows[b], gsem).wait()
        pltpu.async_copy(rows[b], acc_s.at[dsti[bank][b]], ssems[b], add=True)
      if with_deg:
        deg_ops(bank)

    fire_idx(0, 0)

    def pair(p, carry):
      step(2 * p, 0, 1)
      step(2 * p + 1, 1, 0)
      return carry

    lax.fori_loop(0, NSTEP // 2, pair, 0)
    drain_scatters(1)
    plsc.subcore_barrier()

    out0 = c * N_PAD + row0
    bufs = list(rows)
    nring = len(bufs)
    if with_deg:
      pltpu.async_copy(deg_v, deg_out.at[w], isem)
    for i in range(NSEG):
      b = bufs[i % nring]
      if i >= nring:
        pltpu.make_async_copy(b, agg_out.at[pl.ds(out0, K)], ssem).wait()
      pltpu.sync_copy(acc_s.at[pl.ds(row0 + i * K, K)], b)
      pltpu.async_copy(b, agg_out.at[pl.ds(out0 + i * K, K)], ssem)
    for i in range(max(NSEG - nring, 0), NSEG):
      pltpu.make_async_copy(bufs[i % nring], agg_out.at[pl.ds(out0, K)],
                            ssem).wait()
    if with_deg:
      pltpu.make_async_copy(deg_v, deg_out.at[w], isem).wait()

  return pl.kernel(
      body, out_type=out_type, mesh=mesh, scratch_types=scratch,
      compiler_params=pltpu.CompilerParams(needs_layout_passes=False))


@functools.lru_cache(maxsize=None)
def _sc_agg_fn(with_deg):
  return _make_sc_agg(with_deg)


def _sc_agg_deg(*args):
  return _sc_agg_fn(True)(*args)


def _sc_agg(*args):
  return _sc_agg_fn(False)(*args)[0]


def _tc_root_body(x_ref, wr_ref, b_ref, o_ref):
  o_ref[...] = (jnp.dot(x_ref[...], wr_ref[...],
                        preferred_element_type=jnp.float32) + b_ref[...])


def _tc_root(feat, w_root, b, out_w):
  n = feat.shape[0]
  return pl.pallas_call(
      _tc_root_body,
      grid=(G,),
      in_specs=[
          pl.BlockSpec((NB, IN), lambda i: (i, 0)),
          pl.BlockSpec((IN, out_w), lambda i: (0, 0)),
          pl.BlockSpec((1, out_w), lambda i: (0, 0)),
      ],
      out_specs=pl.BlockSpec((NB, out_w), lambda i: (i, 0)),
      out_shape=jax.ShapeDtypeStruct((N_PAD, out_w), jnp.float32),
  )(feat, w_root, b)


def _tc_fin_body(elu, root_ref, aggp_ref, degp_ref, wl_ref, o_ref):
  agg = aggp_ref[0] + aggp_ref[1]
  deg = jnp.sum(degp_ref[...], axis=1, keepdims=True)
  inv = 1.0 / jnp.maximum(deg, 1.0)
  z = root_ref[...] + jnp.dot(agg, wl_ref[...],
                              preferred_element_type=jnp.float32) * inv
  if elu:
    z = jnp.where(z > 0, z, jnp.exp(jnp.minimum(z, 0.0)) - 1.0)
  o_ref[...] = z


def _tc_fin(elu, root, aggp, degp, w_rel, out_w):
  return pl.pallas_call(
      functools.partial(_tc_fin_body, elu),
      grid=(G,),
      in_specs=[
          pl.BlockSpec((NB, out_w), lambda i: (i, 0)),
          pl.BlockSpec((NC, NB, H), lambda i: (0, i, 0)),
          pl.BlockSpec((NB, NW), lambda i: (i, 0)),
          pl.BlockSpec((H, out_w), lambda i: (0, 0)),
      ],
      out_specs=pl.BlockSpec((NB, out_w), lambda i: (i, 0)),
      out_shape=jax.ShapeDtypeStruct((N_PAD, out_w), jnp.float32),
  )(root, aggp, degp, w_rel)


OUTW = 8


def kernel(x, edge_index, edge_type, W1_rel, W1_root, b1, W2_rel, W2_root, b2):
  del edge_type
  src = edge_index[0]
  dst = edge_index[1]
  zeros_f = jnp.zeros((K, H), jnp.float32)
  zeros_deg = jnp.zeros((N_PAD,), jnp.float32)

  aggp1, degp = _sc_agg_deg(x, src, dst, zeros_f, zeros_deg)
  root1 = _tc_root(x, W1_root, b1.reshape(1, H), H)
  aggp1 = aggp1.reshape(NC, N_PAD, H)
  degp = degp.T
  h = _tc_fin(True, root1, aggp1, degp, W1_rel[0], H)

  aggp2 = _sc_agg(h, src, dst, zeros_f, zeros_deg).reshape(NC, N_PAD, H)
  w2_root = jnp.pad(W2_root, ((0, 0), (0, OUTW - OUT)))
  w2_rel = jnp.pad(W2_rel[0], ((0, 0), (0, OUTW - OUT)))
  b2_p = jnp.pad(b2, (0, OUTW - OUT)).reshape(1, OUTW)
  root2 = _tc_root(h, w2_root, b2_p, OUTW)
  out = _tc_fin(False, root2, aggp2, degp, w2_rel, OUTW)
  return out[:N, :OUT]


# speedup vs baseline: 13.2710x; 1.0007x over previous
import functools

import jax
import jax.numpy as jnp
from jax import lax
from jax.experimental import pallas as pl
from jax.experimental.pallas import tpu as pltpu
from jax.experimental.pallas import tpu_sc as plsc

N = 10000
E = 320000
IN = 128
H = 128
OUT = 2

NC = 2
NS = 16
NW = NC * NS
N_PAD = 10240
RPT = N_PAD // NS
EPW = E // NW
K = 40
NCHUNK = EPW // K
NB = 1024
G = N_PAD // NB


RING = 5
NSTEP = NCHUNK // RING


def _make_sc_agg(with_deg):
  mesh = plsc.VectorSubcoreMesh(core_axis_name="c", subcore_axis_name="s")
  out_type = [jax.ShapeDtypeStruct((NC * N_PAD, H), jnp.float32)]
  if with_deg:
    out_type.append(jax.ShapeDtypeStruct((NW, N_PAD), jnp.float32))
  scratch = (
      [pltpu.VMEM((K,), jnp.int32)] * (2 * RING)
      + [pltpu.VMEM((K,), jnp.int32)] * (2 * RING)
      + [pltpu.VMEM((K, H), jnp.float32)] * RING
      + [pltpu.VMEM_SHARED((N_PAD, H), jnp.float32)]
  )
  if with_deg:
    scratch.append(pltpu.VMEM((N_PAD,), jnp.float32))
  scratch += [pltpu.SemaphoreType.DMA] * (3 + 2 * RING)
  NSEG = RPT // K

  def body(feat, src, dst, zeros_f, zeros_deg, *refs):
    if with_deg:
      agg_out, deg_out = refs[0], refs[1]
      rest = refs[2:]
    else:
      agg_out = refs[0]
      rest = refs[1:]
    srci_flat = rest[:2 * RING]
    dsti_flat = rest[2 * RING:4 * RING]
    rows = rest[4 * RING:5 * RING]
    acc_s = rest[5 * RING]
    base = 5 * RING + (2 if with_deg else 1)
    if with_deg:
      deg_v = rest[5 * RING + 1]
    gsem, ssem, isem = rest[base:base + 3]
    ssems = rest[base + 3:base + 3 + RING]
    gsems = rest[base + 3 + RING:base + 3 + 2 * RING]
    srci = [srci_flat[bank * RING:(bank + 1) * RING] for bank in range(2)]
    dsti = [dsti_flat[bank * RING:(bank + 1) * RING] for bank in range(2)]
    c = lax.axis_index("c")
    s = lax.axis_index("s")
    w = c * NS + s
    row0 = s * RPT
    ebase = w * EPW
    stage = rows[0]

    ones16 = jnp.full((16,), 1.0, jnp.float32)
    tail = K % 16
    tail_mask = (lax.iota(jnp.int32, 16) >= 16 - tail) if tail else None

    def fire_idx(t, bank):
      for b in range(RING):
        off = ebase + t * (RING * K) + b * K
        pltpu.async_copy(src.at[pl.ds(off, K)], srci[bank][b], isem)
        pltpu.async_copy(dst.at[pl.ds(off, K)], dsti[bank][b], isem)

    def drain_idx(bank):
      for b in range(RING):
        pltpu.make_async_copy(src.at[pl.ds(0, K)], srci[bank][b], isem).wait()
        pltpu.make_async_copy(dst.at[pl.ds(0, K)], dsti[bank][b], isem).wait()

    def drain_scatters(bank):
      for b in range(RING):
        pltpu.make_async_copy(rows[b], acc_s.at[dsti[bank][b]],
                              ssems[b]).wait()

    def deg_ops(bank):
      for b in range(RING):
        for i in range(K // 16):
          dst16 = dsti[bank][b][pl.ds(i * 16, 16)]
          plsc.addupdate_scatter(deg_v, [dst16], ones16)
        if tail:
          dst16 = dsti[bank][b][pl.ds(K - 16, 16)]
          plsc.addupdate_scatter(deg_v, [dst16], ones16, mask=tail_mask)

    def step(t, bank, prev_bank):
      drain_idx(bank)
      for b in range(RING):
        @pl.when(t > 0)
        def _():
          pltpu.make_async_copy(rows[b], acc_s.at[dsti[prev_bank][b]],
                                ssems[b]).wait()
        pltpu.async_copy(feat.at[srci[bank][b]], rows[b], gsems[b])

      @pl.when(t + 1 < NSTEP)
      def _():
        fire_idx(t + 1, 1 - bank)

      for b in range(RING):
        pltpu.make_async_copy(feat.at[srci[bank][b]], rows[b],
                              gsems[b]).wait()
        pltpu.async_copy(rows[b], acc_s.at[dsti[bank][b]], ssems[b], add=True)
      if with_deg:
        deg_ops(bank)

    fire_idx(0, 0)
    pltpu.sync_copy(zeros_f, stage)
    for i in range(NSEG):
      pltpu.async_copy(stage, acc_s.at[pl.ds(row0 + i * K, K)], gsem)
    if with_deg:
      pltpu.sync_copy(zeros_deg, deg_v)
    drain_idx(0)
    for b in range(1, RING):
      pltpu.async_copy(feat.at[srci[0][b]], rows[b], gsems[b])
    for i in range(NSEG):
      pltpu.make_async_copy(stage, acc_s.at[pl.ds(row0, K)], gsem).wait()
    plsc.subcore_barrier()

    pltpu.async_copy(feat.at[srci[0][0]], rows[0], gsems[0])
    fire_idx(1, 1)
    for b in range(RING):
      pltpu.make_async_copy(feat.at[srci[0][b]], rows[b], gsems[b]).wait()
      pltpu.async_copy(rows[b], acc_s.at[dsti[0][b]], ssems[b], add=True)
    if with_deg:
      deg_ops(0)
    step(1, 1, 0)

    def pair(p, carry):
      step(2 * p, 0, 1)
      step(2 * p + 1, 1, 0)
      return carry

    lax.fori_loop(1, NSTEP // 2, pair, 0)
    drain_scatters(1)
    plsc.subcore_barrier()

    out0 = c * N_PAD + row0
    bufs = list(rows)
    nring = len(bufs)
    if with_deg:
      pltpu.async_copy(deg_v, deg_out.at[w], isem)
    for i in range(NSEG):
      b = bufs[i % nring]
      if i >= nring:
        pltpu.make_async_copy(b, agg_out.at[pl.ds(out0, K)], ssem).wait()
      pltpu.sync_copy(acc_s.at[pl.ds(row0 + i * K, K)], b)
      pltpu.async_copy(b, agg_out.at[pl.ds(out0 + i * K, K)], ssem)
    for i in range(max(NSEG - nring, 0), NSEG):
      pltpu.make_async_copy(bufs[i % nring], agg_out.at[pl.ds(out0, K)],
                            ssem).wait()
    if with_deg:
      pltpu.make_async_copy(deg_v, deg_out.at[w], isem).wait()

  return pl.kernel(
      body, out_type=out_type, mesh=mesh, scratch_types=scratch,
      compiler_params=pltpu.CompilerParams(needs_layout_passes=False))


@functools.lru_cache(maxsize=None)
def _sc_agg_fn(with_deg):
  return _make_sc_agg(with_deg)


def _sc_agg_deg(*args):
  return _sc_agg_fn(True)(*args)


def _sc_agg(*args):
  return _sc_agg_fn(False)(*args)[0]


def _tc_root_body(x_ref, wr_ref, b_ref, o_ref):
  o_ref[...] = (jnp.dot(x_ref[...], wr_ref[...],
                        preferred_element_type=jnp.float32) + b_ref[...])


def _tc_root(feat, w_root, b, out_w):
  n = feat.shape[0]
  return pl.pallas_call(
      _tc_root_body,
      grid=(G,),
      in_specs=[
          pl.BlockSpec((NB, IN), lambda i: (i, 0)),
          pl.BlockSpec((IN, out_w), lambda i: (0, 0)),
          pl.BlockSpec((1, out_w), lambda i: (0, 0)),
      ],
      out_specs=pl.BlockSpec((NB, out_w), lambda i: (i, 0)),
      out_shape=jax.ShapeDtypeStruct((N_PAD, out_w), jnp.float32),
  )(feat, w_root, b)


def _tc_fin_body(elu, root_ref, aggp_ref, degp_ref, wl_ref, o_ref):
  agg = aggp_ref[0] + aggp_ref[1]
  deg = jnp.sum(degp_ref[...], axis=1, keepdims=True)
  inv = 1.0 / jnp.maximum(deg, 1.0)
  z = root_ref[...] + jnp.dot(agg, wl_ref[...],
                              preferred_element_type=jnp.float32) * inv
  if elu:
    z = jnp.where(z > 0, z, jnp.exp(jnp.minimum(z, 0.0)) - 1.0)
  o_ref[...] = z


def _tc_fin(elu, root, aggp, degp, w_rel, out_w):
  return pl.pallas_call(
      functools.partial(_tc_fin_body, elu),
      grid=(G,),
      in_specs=[
          pl.BlockSpec((NB, out_w), lambda i: (i, 0)),
          pl.BlockSpec((NC, NB, H), lambda i: (0, i, 0)),
          pl.BlockSpec((NB, NW), lambda i: (i, 0)),
          pl.BlockSpec((H, out_w), lambda i: (0, 0)),
      ],
      out_specs=pl.BlockSpec((NB, out_w), lambda i: (i, 0)),
      out_shape=jax.ShapeDtypeStruct((N_PAD, out_w), jnp.float32),
  )(root, aggp, degp, w_rel)


OUTW = 8


def kernel(x, edge_index, edge_type, W1_rel, W1_root, b1, W2_rel, W2_root, b2):
  del edge_type
  src = edge_index[0]
  dst = edge_index[1]
  zeros_f = jnp.zeros((K, H), jnp.float32)
  zeros_deg = jnp.zeros((N_PAD,), jnp.float32)

  aggp1, degp = _sc_agg_deg(x, src, dst, zeros_f, zeros_deg)
  root1 = _tc_root(x, W1_root, b1.reshape(1, H), H)
  aggp1 = aggp1.reshape(NC, N_PAD, H)
  degp = degp.T
  h = _tc_fin(True, root1, aggp1, degp, W1_rel[0], H)

  aggp2 = _sc_agg(h, src, dst, zeros_f, zeros_deg).reshape(NC, N_PAD, H)
  w2_root = jnp.pad(W2_root, ((0, 0), (0, OUTW - OUT)))
  w2_rel = jnp.pad(W2_rel[0], ((0, 0), (0, OUTW - OUT)))
  b2_p = jnp.pad(b2, (0, OUTW - OUT)).reshape(1, OUTW)
  root2 = _tc_root(h, w2_root, b2_p, OUTW)
  out = _tc_fin(False, root2, aggp2, degp, w2_rel, OUTW)
  return out[:N, :OUT]
